# Initial kernel scaffold; baseline (speedup 1.0000x reference)
import jax
import jax.numpy as jnp
from jax.experimental import pallas as pl


def kernel(receivers, senders, sender_idx, edge_attribute, W_source, W_target, W_edge, attn):
    raise NotImplementedError("write your pallas kernel here")


# speedup vs baseline: 16.6049x; 16.6049x over previous
import functools

import jax
import jax.numpy as jnp
from jax import lax
from jax.experimental import pallas as pl
from jax.experimental.pallas import tpu as pltpu
from jax.experimental.pallas import tpu_sc as plsc

N_NODES = 10000
N_EDGES = 320000
IN_C = 128
OUT_C = 16
HEADS = 4
HC = OUT_C * HEADS


EDGE_B = 1600


def _s1_body(r_ref, s_ref, ea_ref, wt_ref, ws_ref, we_ref, am_ref, bm_ref,
             p_ref, ez_ref):
    t = jnp.dot(r_ref[...], wt_ref[...])
    s = jnp.dot(s_ref[...], ws_ref[...])
    e = jnp.dot(ea_ref[...], we_ref[...])
    v = s + t + e
    v = jnp.where(v > 0, v, 0.01 * v)
    z = jnp.dot(v, am_ref[...], precision=jax.lax.Precision.HIGHEST)
    ez = jnp.exp(z)
    p_ref[...] = t * jnp.dot(ez, bm_ref[...],
                              precision=jax.lax.Precision.HIGHEST)
    ez_ref[...] = ez


def _stage1(receivers, senders, edge_attribute, W_target, W_source, W_edge,
            attn_mat, bcast_mat):
    nblk = N_EDGES // EDGE_B
    full = lambda shape: pl.BlockSpec(shape, lambda i: (0, 0))
    edge_blk = lambda w: pl.BlockSpec((EDGE_B, w), lambda i: (i, 0))
    return pl.pallas_call(
        _s1_body,
        grid=(nblk,),
        in_specs=[
            edge_blk(IN_C), edge_blk(IN_C), edge_blk(IN_C),
            full((IN_C, HC)), full((IN_C, HC)), full((IN_C, HC)),
            full((HC, HEADS)), full((HEADS, HC)),
        ],
        out_specs=[edge_blk(HC), edge_blk(HEADS)],
        out_shape=[
            jax.ShapeDtypeStruct((N_EDGES, HC), jnp.float32),
            jax.ShapeDtypeStruct((N_EDGES, HEADS), jnp.float32),
        ],
    )(receivers, senders, edge_attribute, W_target, W_source, W_edge,
      attn_mat, bcast_mat)


NODE_B = 128
CHUNK_B = 512


def _s2_body(c0_ref, ntrip_ref,
             p_hbm, ez_hbm, idxf_hbm,
             sel_ref, bm_ref,
             aggr_ref, recip_ref,
             p_buf, ez_buf, idx_buf, acc_n, acc_d, sem):
    n = pl.program_id(0)
    node_base = (n * NODE_B).astype(jnp.float32)
    node_ids = node_base + lax.broadcasted_iota(
        jnp.int32, (NODE_B, 1), 0).astype(jnp.float32)

    acc_n[...] = jnp.zeros_like(acc_n)
    acc_d[...] = jnp.zeros_like(acc_d)

    c0 = c0_ref[n]
    ntrip = ntrip_ref[n]

    def trip(j, _):
        c = c0 + j
        cp_p = pltpu.make_async_copy(
            p_hbm.at[pl.ds(c * CHUNK_B, CHUNK_B), :], p_buf, sem)
        cp_e = pltpu.make_async_copy(
            ez_hbm.at[pl.ds(c * CHUNK_B, CHUNK_B), :], ez_buf, sem)
        cp_i = pltpu.make_async_copy(
            idxf_hbm.at[:, pl.ds(c * CHUNK_B, CHUNK_B)], idx_buf, sem)
        cp_p.start(); cp_e.start(); cp_i.start()
        cp_p.wait(); cp_e.wait(); cp_i.wait()
        onehot = (idx_buf[...] == node_ids).astype(jnp.float32)
        acc_n[...] += jnp.dot(onehot, p_buf[...],
                              precision=jax.lax.Precision.HIGHEST)
        acc_d[...] += jnp.dot(onehot, ez_buf[...],
                              precision=jax.lax.Precision.HIGHEST)
        return 0

    lax.fori_loop(0, ntrip, trip, 0)

    recip = 1.0 / (acc_d[...] + 1e-16)
    rb = jnp.dot(recip, bm_ref[...], precision=jax.lax.Precision.HIGHEST)
    aggr_ref[...] = jnp.dot(acc_n[...] * rb, sel_ref[...],
                            precision=jax.lax.Precision.HIGHEST)
    recip_ref[...] = recip


def _stage2(p, ez, idx_f32_row, chunk0, ntrips, sel_mat, bcast_mat):
    nblk = N_NODES // NODE_B + (1 if N_NODES % NODE_B else 0)
    grid_spec = pltpu.PrefetchScalarGridSpec(
        num_scalar_prefetch=2,
        grid=(nblk,),
        in_specs=[
            pl.BlockSpec(memory_space=pl.ANY),
            pl.BlockSpec(memory_space=pl.ANY),
            pl.BlockSpec(memory_space=pl.ANY),
            pl.BlockSpec((HC, OUT_C), lambda n, c0, nt: (0, 0)),
            pl.BlockSpec((HEADS, HC), lambda n, c0, nt: (0, 0)),
        ],
        out_specs=[
            pl.BlockSpec((NODE_B, OUT_C), lambda n, c0, nt: (n, 0)),
            pl.BlockSpec((NODE_B, HEADS), lambda n, c0, nt: (n, 0)),
        ],
        scratch_shapes=[
            pltpu.VMEM((CHUNK_B, HC), jnp.float32),
            pltpu.VMEM((CHUNK_B, HEADS), jnp.float32),
            pltpu.VMEM((1, CHUNK_B), jnp.float32),
            pltpu.VMEM((NODE_B, HC), jnp.float32),
            pltpu.VMEM((NODE_B, HEADS), jnp.float32),
            pltpu.SemaphoreType.DMA,
        ],
    )
    npad = nblk * NODE_B
    return pl.pallas_call(
        _s2_body,
        grid_spec=grid_spec,
        out_shape=[
            jax.ShapeDtypeStruct((npad, OUT_C), jnp.float32),
            jax.ShapeDtypeStruct((npad, HEADS), jnp.float32),
        ],
    )(chunk0, ntrips, p, ez, idx_f32_row, sel_mat, bcast_mat)


def _sc_gather(table_flat, idx_pad, per_tile):
    info = plsc.get_sparse_core_info()
    nc = info.num_cores
    ep = idx_pad.shape[0]
    tbl = table_flat.shape[0]

    mesh = plsc.VectorSubcoreMesh(core_axis_name="c", subcore_axis_name="s")

    @functools.partial(
        pl.kernel, mesh=mesh,
        out_type=jax.ShapeDtypeStruct((ep * HEADS,), jnp.float32),
        compiler_params=pltpu.CompilerParams(needs_layout_passes=False),
        scratch_types=[
            pltpu.VMEM((tbl,), jnp.float32),
            pltpu.VMEM((per_tile,), jnp.int32),
            pltpu.VMEM((per_tile * HEADS,), jnp.float32),
        ],
    )
    def k(table_hbm, idx_hbm, out_hbm, tbl_v, idx_v, out_v):
        wid = lax.axis_index("s") * nc + lax.axis_index("c")
        base = wid * per_tile
        pltpu.sync_copy(table_hbm, tbl_v)
        pltpu.sync_copy(idx_hbm.at[pl.ds(base, per_tile)], idx_v)
        lane = lax.iota(jnp.int32, 16) * HEADS

        def it(i, _):
            flat = idx_v[pl.ds(i * 16, 16)] * HEADS
            off = i * (16 * HEADS)
            for h in range(HEADS):
                g = plsc.load_gather(tbl_v, [flat + h])
                plsc.store_scatter(out_v, [lane + (off + h)], g)
            return 0

        lax.fori_loop(0, per_tile // 16, it, 0)
        pltpu.sync_copy(out_v, out_hbm.at[pl.ds(base * HEADS, per_tile * HEADS)])

    return k(table_flat, idx_pad)


EDGE_B4 = 1600


def _s4_body(p_ref, rp_ref, bm_ref, sel_ref, out_ref):
    rb = jnp.dot(rp_ref[...], bm_ref[...], precision=jax.lax.Precision.HIGHEST)
    out_ref[...] = jnp.dot(p_ref[...] * rb, sel_ref[...],
                           precision=jax.lax.Precision.HIGHEST)


def _stage4(p, recip_pe, bcast_mat, sel_mat):
    nblk = N_EDGES // EDGE_B4
    return pl.pallas_call(
        _s4_body,
        grid=(nblk,),
        in_specs=[
            pl.BlockSpec((EDGE_B4, HC), lambda i: (i, 0)),
            pl.BlockSpec((EDGE_B4, HEADS), lambda i: (i, 0)),
            pl.BlockSpec((HEADS, HC), lambda i: (0, 0)),
            pl.BlockSpec((HC, OUT_C), lambda i: (0, 0)),
        ],
        out_specs=pl.BlockSpec((EDGE_B4, OUT_C), lambda i: (i, 0)),
        out_shape=jax.ShapeDtypeStruct((N_EDGES, OUT_C), jnp.float32),
    )(p, recip_pe, bcast_mat, sel_mat)


def kernel(receivers, senders, sender_idx, edge_attribute, W_source,
           W_target, W_edge, attn):
    idx = sender_idx.astype(jnp.int32)

    a0 = attn.reshape(HEADS, OUT_C)
    attn_mat = (a0[:, :, None] * jnp.eye(HEADS, dtype=jnp.float32)[:, None, :]
                ).reshape(HC, HEADS)
    bcast_mat = (jnp.eye(HEADS, dtype=jnp.float32)[:, :, None]
                 * jnp.ones((1, 1, OUT_C), jnp.float32)).reshape(HEADS, HC)
    sel_mat = jnp.tile(jnp.eye(OUT_C, dtype=jnp.float32) * (1.0 / HEADS),
                       (HEADS, 1))

    p, ez = _stage1(receivers, senders, edge_attribute, W_target, W_source,
                    W_edge, attn_mat, bcast_mat)

    nblk = N_NODES // NODE_B + (1 if N_NODES % NODE_B else 0)
    bounds = jnp.searchsorted(idx, jnp.arange(nblk + 1, dtype=jnp.int32) * NODE_B)
    c0 = bounds[:-1] // CHUNK_B
    c1 = (bounds[1:] + CHUNK_B - 1) // CHUNK_B
    ntrips = (c1 - c0).astype(jnp.int32)
    c0 = c0.astype(jnp.int32)

    idx_f32_row = idx.astype(jnp.float32).reshape(1, N_EDGES)

    aggr, recip = _stage2(p, ez, idx_f32_row, c0, ntrips, sel_mat, bcast_mat)

    ntiles = 32
    per_tile = -(-N_EDGES // ntiles)
    per_tile = -(-per_tile // 128) * 128
    ep = ntiles * per_tile
    idx_pad = jnp.pad(idx, (0, ep - N_EDGES))
    rp_flat = _sc_gather(recip.reshape(-1), idx_pad, per_tile)
    rp = rp_flat.reshape(ep, HEADS)

    m_out = _stage4(p, rp, bcast_mat, sel_mat)
    return (aggr[:N_NODES], m_out)


# speedup vs baseline: 26.3111x; 1.5845x over previous
import functools

import jax
import jax.numpy as jnp
from jax import lax
from jax.experimental import pallas as pl
from jax.experimental.pallas import tpu as pltpu
from jax.experimental.pallas import tpu_sc as plsc

N_NODES = 10000
N_EDGES = 320000
IN_C = 128
OUT_C = 16
HEADS = 4
HC = OUT_C * HEADS


CHUNK = 2560
WIN = 128
NPAD = 10240


def _s12_body(base_ref, nwin_ref,
              r_ref, s_ref, ea_ref, idx_ref,
              wt_ref, ws_ref, we_ref, am_ref, bm_ref, sel_ref,
              p_ref, aggr_ref, recip_ref,
              num_acc, den_acc):
    i = pl.program_id(0)
    nsteps = pl.num_programs(0)

    @pl.when(i == 0)
    def _init():
        num_acc[...] = jnp.zeros_like(num_acc)
        den_acc[...] = jnp.zeros_like(den_acc)

    t = jnp.dot(r_ref[...], wt_ref[...])
    s = jnp.dot(s_ref[...], ws_ref[...])
    e = jnp.dot(ea_ref[...], we_ref[...])
    v = s + t + e
    v = jnp.where(v > 0, v, 0.01 * v)
    z = jnp.dot(v, am_ref[...], precision=jax.lax.Precision.HIGHEST)
    ez = jnp.exp(z)
    p = t * jnp.dot(ez, bm_ref[...], precision=jax.lax.Precision.HIGHEST)
    p_ref[...] = p

    base = base_ref[i]
    nwin = nwin_ref[i]
    idx_row = idx_ref[...]

    def win(w, _):
        w0 = base + w * WIN
        nid = w0.astype(jnp.float32) + lax.broadcasted_iota(
            jnp.int32, (WIN, 1), 0).astype(jnp.float32)
        oh = (idx_row == nid).astype(jnp.float32)
        num_acc[pl.ds(w0, WIN), :] += jnp.dot(
            oh, p, precision=jax.lax.Precision.HIGHEST)
        den_acc[pl.ds(w0, WIN), :] += jnp.dot(
            oh, ez, precision=jax.lax.Precision.HIGHEST)
        return 0

    lax.fori_loop(0, nwin, win, 0)

    @pl.when(i == nsteps - 1)
    def _epilogue():
        recip = 1.0 / (den_acc[...] + 1e-16)
        recip_ref[...] = recip
        rb = jnp.dot(recip, bm_ref[...], precision=jax.lax.Precision.HIGHEST)
        aggr_ref[...] = jnp.dot(num_acc[...] * rb, sel_ref[...],
                                precision=jax.lax.Precision.HIGHEST)


def _stage12(receivers, senders, edge_attribute, idx_f32_row, bases, nwins,
             W_target, W_source, W_edge, attn_mat, bcast_mat, sel_mat):
    nblk = N_EDGES // CHUNK
    full = lambda shape: pl.BlockSpec(shape, lambda i, b, nw: (0, 0))
    edge_blk = lambda w: pl.BlockSpec((CHUNK, w), lambda i, b, nw: (i, 0))
    grid_spec = pltpu.PrefetchScalarGridSpec(
        num_scalar_prefetch=2,
        grid=(nblk,),
        in_specs=[
            edge_blk(IN_C), edge_blk(IN_C), edge_blk(IN_C),
            pl.BlockSpec((1, CHUNK), lambda i, b, nw: (0, i)),
            full((IN_C, HC)), full((IN_C, HC)), full((IN_C, HC)),
            full((HC, HEADS)), full((HEADS, HC)), full((HC, OUT_C)),
        ],
        out_specs=[
            edge_blk(HC),
            pl.BlockSpec((NPAD, OUT_C), lambda i, b, nw: (0, 0)),
            pl.BlockSpec((NPAD, HEADS), lambda i, b, nw: (0, 0)),
        ],
        scratch_shapes=[
            pltpu.VMEM((NPAD, HC), jnp.float32),
            pltpu.VMEM((NPAD, HEADS), jnp.float32),
        ],
    )
    return pl.pallas_call(
        _s12_body,
        grid_spec=grid_spec,
        out_shape=[
            jax.ShapeDtypeStruct((N_EDGES, HC), jnp.float32),
            jax.ShapeDtypeStruct((NPAD, OUT_C), jnp.float32),
            jax.ShapeDtypeStruct((NPAD, HEADS), jnp.float32),
        ],
    )(bases, nwins, receivers, senders, edge_attribute, idx_f32_row,
      W_target, W_source, W_edge, attn_mat, bcast_mat, sel_mat)


def _sc_gather(table_flat, idx_pad, per_tile):
    info = plsc.get_sparse_core_info()
    nc = info.num_cores
    ep = idx_pad.shape[0]
    tbl = table_flat.shape[0]

    mesh = plsc.VectorSubcoreMesh(core_axis_name="c", subcore_axis_name="s")

    @functools.partial(
        pl.kernel, mesh=mesh,
        out_type=jax.ShapeDtypeStruct((ep * HEADS,), jnp.float32),
        compiler_params=pltpu.CompilerParams(needs_layout_passes=False),
        scratch_types=[
            pltpu.VMEM((tbl,), jnp.float32),
            pltpu.VMEM((per_tile,), jnp.int32),
            pltpu.VMEM((per_tile * HEADS,), jnp.float32),
        ],
    )
    def k(table_hbm, idx_hbm, out_hbm, tbl_v, idx_v, out_v):
        wid = lax.axis_index("s") * nc + lax.axis_index("c")
        base = wid * per_tile
        pltpu.sync_copy(table_hbm, tbl_v)
        pltpu.sync_copy(idx_hbm.at[pl.ds(base, per_tile)], idx_v)
        lane = lax.iota(jnp.int32, 16) * HEADS

        def it(i, _):
            flat = idx_v[pl.ds(i * 16, 16)] * HEADS
            off = i * (16 * HEADS)
            for h in range(HEADS):
                g = plsc.load_gather(tbl_v, [flat + h])
                plsc.store_scatter(out_v, [lane + (off + h)], g)
            return 0

        lax.fori_loop(0, per_tile // 16, it, 0)
        pltpu.sync_copy(out_v, out_hbm.at[pl.ds(base * HEADS, per_tile * HEADS)])

    return k(table_flat, idx_pad)


EDGE_B4 = 1600


def _s4_body(p_ref, rp_ref, bm_ref, sel_ref, out_ref):
    rb = jnp.dot(rp_ref[...], bm_ref[...], precision=jax.lax.Precision.HIGHEST)
    out_ref[...] = jnp.dot(p_ref[...] * rb, sel_ref[...],
                           precision=jax.lax.Precision.HIGHEST)


def _stage4(p, recip_pe, bcast_mat, sel_mat):
    nblk = N_EDGES // EDGE_B4
    return pl.pallas_call(
        _s4_body,
        grid=(nblk,),
        in_specs=[
            pl.BlockSpec((EDGE_B4, HC), lambda i: (i, 0)),
            pl.BlockSpec((EDGE_B4, HEADS), lambda i: (i, 0)),
            pl.BlockSpec((HEADS, HC), lambda i: (0, 0)),
            pl.BlockSpec((HC, OUT_C), lambda i: (0, 0)),
        ],
        out_specs=pl.BlockSpec((EDGE_B4, OUT_C), lambda i: (i, 0)),
        out_shape=jax.ShapeDtypeStruct((N_EDGES, OUT_C), jnp.float32),
    )(p, recip_pe, bcast_mat, sel_mat)


def kernel(receivers, senders, sender_idx, edge_attribute, W_source,
           W_target, W_edge, attn):
    idx = sender_idx.astype(jnp.int32)

    a0 = attn.reshape(HEADS, OUT_C)
    attn_mat = (a0[:, :, None] * jnp.eye(HEADS, dtype=jnp.float32)[:, None, :]
                ).reshape(HC, HEADS)
    bcast_mat = (jnp.eye(HEADS, dtype=jnp.float32)[:, :, None]
                 * jnp.ones((1, 1, OUT_C), jnp.float32)).reshape(HEADS, HC)
    sel_mat = jnp.tile(jnp.eye(OUT_C, dtype=jnp.float32) * (1.0 / HEADS),
                       (HEADS, 1))

    starts = idx[::CHUNK]
    ends = idx[CHUNK - 1::CHUNK]
    bases = (starts // 8) * 8
    nwins = (ends - bases + WIN) // WIN

    idx_f32_row = idx.astype(jnp.float32).reshape(1, N_EDGES)

    p, aggr, recip = _stage12(
        receivers, senders, edge_attribute, idx_f32_row, bases, nwins,
        W_target, W_source, W_edge, attn_mat, bcast_mat, sel_mat)

    ntiles = 32
    per_tile = -(-N_EDGES // ntiles)
    per_tile = -(-per_tile // 128) * 128
    ep = ntiles * per_tile
    idx_pad = jnp.pad(idx, (0, ep - N_EDGES))
    rp_flat = _sc_gather(recip.reshape(-1)[:N_NODES * HEADS], idx_pad, per_tile)
    rp = rp_flat.reshape(ep, HEADS)

    m_out = _stage4(p, rp, bcast_mat, sel_mat)
    return (aggr[:N_NODES], m_out)


# speedup vs baseline: 56.9513x; 2.1645x over previous
import functools

import jax
import jax.numpy as jnp
from jax import lax
from jax.experimental import pallas as pl
from jax.experimental.pallas import tpu as pltpu
from jax.experimental.pallas import tpu_sc as plsc

N_NODES = 10000
N_EDGES = 320000
IN_C = 128
OUT_C = 16
HEADS = 4
HC = OUT_C * HEADS


CHUNK = 2560
WIN = 128
NPAD = 10240


def _s12_body(base_ref, nwin_ref,
              r_ref, s_ref, ea_ref, idx_ref,
              wt_ref, ws_ref, we_ref, am_ref, bm_ref,
              bm32_ref, sel32_ref,
              p_ref, aggr_ref, recip_ref,
              num_acc, den_acc):
    i = pl.program_id(0)
    nsteps = pl.num_programs(0)

    @pl.when(i == 0)
    def _init():
        num_acc[...] = jnp.zeros_like(num_acc)
        den_acc[...] = jnp.zeros_like(den_acc)

    f32 = jnp.float32
    dotf = functools.partial(jnp.dot, preferred_element_type=f32)
    t = dotf(r_ref[...].astype(jnp.bfloat16), wt_ref[...])
    s = dotf(s_ref[...].astype(jnp.bfloat16), ws_ref[...])
    e = dotf(ea_ref[...].astype(jnp.bfloat16), we_ref[...])
    v = s + t + e
    v = jnp.where(v > 0, v, 0.01 * v)
    z = jnp.dot(v, am_ref[...])
    ez = jnp.exp(z)
    ez16 = ez.astype(jnp.bfloat16)
    p = t * jnp.dot(ez, bm32_ref[...])
    p_ref[...] = p

    base = base_ref[i]
    nwin = nwin_ref[i]
    idx_row = idx_ref[...]
    p16 = p.astype(jnp.bfloat16)

    def win(w, _):
        w0 = base + w * WIN
        nid = w0.astype(jnp.float32) + lax.broadcasted_iota(
            jnp.int32, (WIN, 1), 0).astype(jnp.float32)
        oh = (idx_row == nid).astype(jnp.bfloat16)
        num_acc[pl.ds(w0, WIN), :] += dotf(oh, p16)
        den_acc[pl.ds(w0, WIN), :] += dotf(oh, ez16)
        return 0

    lax.fori_loop(0, nwin, win, 0)

    @pl.when(i == nsteps - 1)
    def _epilogue():
        recip = 1.0 / (den_acc[...] + 1e-16)
        recip_ref[...] = recip
        rb = jnp.dot(recip, bm32_ref[...])
        aggr_ref[...] = jnp.dot(num_acc[...] * rb, sel32_ref[...])


def _stage12(receivers, senders, edge_attribute, idx_f32_row, bases, nwins,
             W_target, W_source, W_edge, attn_mat, bcast_mat, bcast32,
             sel32):
    nblk = N_EDGES // CHUNK
    full = lambda shape: pl.BlockSpec(shape, lambda i, b, nw: (0, 0))
    edge_blk = lambda w: pl.BlockSpec((CHUNK, w), lambda i, b, nw: (i, 0))
    grid_spec = pltpu.PrefetchScalarGridSpec(
        num_scalar_prefetch=2,
        grid=(nblk,),
        in_specs=[
            edge_blk(IN_C), edge_blk(IN_C), edge_blk(IN_C),
            pl.BlockSpec((1, CHUNK), lambda i, b, nw: (0, i)),
            full((IN_C, HC)), full((IN_C, HC)), full((IN_C, HC)),
            full((HC, HEADS)), full((HEADS, HC)),
            full((HEADS, HC)), full((HC, OUT_C)),
        ],
        out_specs=[
            edge_blk(HC),
            pl.BlockSpec((NPAD, OUT_C), lambda i, b, nw: (0, 0)),
            pl.BlockSpec((NPAD, HEADS), lambda i, b, nw: (0, 0)),
        ],
        scratch_shapes=[
            pltpu.VMEM((NPAD, HC), jnp.float32),
            pltpu.VMEM((NPAD, HEADS), jnp.float32),
        ],
    )
    return pl.pallas_call(
        _s12_body,
        grid_spec=grid_spec,
        out_shape=[
            jax.ShapeDtypeStruct((N_EDGES, HC), jnp.float32),
            jax.ShapeDtypeStruct((NPAD, OUT_C), jnp.float32),
            jax.ShapeDtypeStruct((NPAD, HEADS), jnp.float32),
        ],
    )(bases, nwins, receivers, senders, edge_attribute, idx_f32_row,
      W_target, W_source, W_edge, attn_mat, bcast_mat, bcast32, sel32)


def _sc_gather(table_flat, idx_pad, per_tile):
    info = plsc.get_sparse_core_info()
    nc = info.num_cores
    ep = idx_pad.shape[0]
    tbl = table_flat.shape[0]

    mesh = plsc.VectorSubcoreMesh(core_axis_name="c", subcore_axis_name="s")

    @functools.partial(
        pl.kernel, mesh=mesh,
        out_type=jax.ShapeDtypeStruct((ep * HEADS,), jnp.float32),
        compiler_params=pltpu.CompilerParams(needs_layout_passes=False),
        scratch_types=[
            pltpu.VMEM((tbl,), jnp.float32),
            pltpu.VMEM((per_tile,), jnp.int32),
            pltpu.VMEM((per_tile * HEADS,), jnp.float32),
        ],
    )
    def k(table_hbm, idx_hbm, out_hbm, tbl_v, idx_v, out_v):
        wid = lax.axis_index("s") * nc + lax.axis_index("c")
        base = wid * per_tile
        pltpu.sync_copy(table_hbm, tbl_v)
        pltpu.sync_copy(idx_hbm.at[pl.ds(base, per_tile)], idx_v)
        lane = lax.iota(jnp.int32, 16) * HEADS

        def it(i, _):
            flat = idx_v[pl.ds(i * 16, 16)] * HEADS
            off = i * (16 * HEADS)
            for h in range(HEADS):
                g = plsc.load_gather(tbl_v, [flat + h])
                plsc.store_scatter(out_v, [lane + (off + h)], g)
            return 0

        lax.fori_loop(0, per_tile // 16, it, 0)
        pltpu.sync_copy(out_v, out_hbm.at[pl.ds(base * HEADS, per_tile * HEADS)])

    return k(table_flat, idx_pad)


EDGE_B4 = 3200


def _s4_body(p_ref, rp_ref, bm_ref, sel_ref, out_ref):
    dotf = functools.partial(jnp.dot, preferred_element_type=jnp.float32)
    rb = dotf(rp_ref[...].astype(jnp.bfloat16), bm_ref[...])
    out_ref[...] = dotf((p_ref[...] * rb).astype(jnp.bfloat16), sel_ref[...])


def _stage4(p, recip_pe, bcast_mat, sel_mat):
    nblk = N_EDGES // EDGE_B4
    return pl.pallas_call(
        _s4_body,
        grid=(nblk,),
        in_specs=[
            pl.BlockSpec((EDGE_B4, HC), lambda i: (i, 0)),
            pl.BlockSpec((EDGE_B4, HEADS), lambda i: (i, 0)),
            pl.BlockSpec((HEADS, HC), lambda i: (0, 0)),
            pl.BlockSpec((HC, OUT_C), lambda i: (0, 0)),
        ],
        out_specs=pl.BlockSpec((EDGE_B4, OUT_C), lambda i: (i, 0)),
        out_shape=jax.ShapeDtypeStruct((N_EDGES, OUT_C), jnp.float32),
    )(p, recip_pe, bcast_mat, sel_mat)


def kernel(receivers, senders, sender_idx, edge_attribute, W_source,
           W_target, W_edge, attn):
    idx = sender_idx.astype(jnp.int32)

    a0 = attn.reshape(HEADS, OUT_C)
    attn_mat = (a0[:, :, None] * jnp.eye(HEADS, dtype=jnp.float32)[:, None, :]
                ).reshape(HC, HEADS)
    bcast32 = (jnp.eye(HEADS, dtype=jnp.float32)[:, :, None]
               * jnp.ones((1, 1, OUT_C), jnp.float32)).reshape(HEADS, HC)
    bcast_mat = bcast32.astype(jnp.bfloat16)
    sel32 = jnp.tile(jnp.eye(OUT_C, dtype=jnp.float32) * (1.0 / HEADS),
                     (HEADS, 1))
    sel_mat = sel32.astype(jnp.bfloat16)
    wt16 = W_target.astype(jnp.bfloat16)
    ws16 = W_source.astype(jnp.bfloat16)
    we16 = W_edge.astype(jnp.bfloat16)

    starts = idx[::CHUNK]
    ends = idx[CHUNK - 1::CHUNK]
    bases = (starts // 8) * 8
    nwins = (ends - bases + WIN) // WIN

    idx_f32_row = idx.astype(jnp.float32).reshape(1, N_EDGES)

    p, aggr, recip = _stage12(
        receivers, senders, edge_attribute, idx_f32_row, bases, nwins,
        wt16, ws16, we16, attn_mat, bcast_mat, bcast32, sel32)

    ntiles = 32
    per_tile = -(-N_EDGES // ntiles)
    per_tile = -(-per_tile // 128) * 128
    ep = ntiles * per_tile
    idx_pad = jnp.pad(idx, (0, ep - N_EDGES))
    rp_flat = _sc_gather(recip.reshape(-1)[:N_NODES * HEADS], idx_pad, per_tile)
    rp = rp_flat.reshape(ep, HEADS)

    m_out = _stage4(p, rp, bcast_mat, sel_mat)
    return (aggr[:N_NODES], m_out)


# speedup vs baseline: 58.7085x; 1.0309x over previous
import functools

import jax
import jax.numpy as jnp
from jax import lax
from jax.experimental import pallas as pl
from jax.experimental.pallas import tpu as pltpu
from jax.experimental.pallas import tpu_sc as plsc

N_NODES = 10000
N_EDGES = 320000
IN_C = 128
OUT_C = 16
HEADS = 4
HC = OUT_C * HEADS


CHUNK = 6400
WIN = 128
NPAD = 10240


def _s12_body(base_ref, nwin_ref,
              r_ref, s_ref, ea_ref, idx_ref,
              wt_ref, ws_ref, we_ref, am_ref, bm_ref,
              bm32_ref, sel32_ref,
              p_ref, aggr_ref, recip_ref,
              num_acc, den_acc):
    i = pl.program_id(0)
    nsteps = pl.num_programs(0)

    @pl.when(i == 0)
    def _init():
        num_acc[...] = jnp.zeros_like(num_acc)
        den_acc[...] = jnp.zeros_like(den_acc)

    f32 = jnp.float32
    dotf = functools.partial(jnp.dot, preferred_element_type=f32)
    t = dotf(r_ref[...].astype(jnp.bfloat16), wt_ref[...])
    s = dotf(s_ref[...].astype(jnp.bfloat16), ws_ref[...])
    e = dotf(ea_ref[...].astype(jnp.bfloat16), we_ref[...])
    v = s + t + e
    v = jnp.where(v > 0, v, 0.01 * v)
    z = jnp.dot(v, am_ref[...])
    ez = jnp.exp(z)
    ez16 = ez.astype(jnp.bfloat16)
    p = t * jnp.dot(ez, bm32_ref[...])
    p_ref[...] = p

    base = base_ref[i]
    nwin = nwin_ref[i]
    idx_row = idx_ref[...]
    p16 = p.astype(jnp.bfloat16)

    def win(w, _):
        w0 = base + w * WIN
        nid = w0.astype(jnp.float32) + lax.broadcasted_iota(
            jnp.int32, (WIN, 1), 0).astype(jnp.float32)
        oh = (idx_row == nid).astype(jnp.bfloat16)
        num_acc[pl.ds(w0, WIN), :] += dotf(oh, p16)
        den_acc[pl.ds(w0, WIN), :] += dotf(oh, ez16)
        return 0

    lax.fori_loop(0, nwin, win, 0)

    @pl.when(i == nsteps - 1)
    def _epilogue():
        recip = 1.0 / (den_acc[...] + 1e-16)
        recip_ref[...] = recip
        rb = jnp.dot(recip, bm32_ref[...])
        aggr_ref[...] = jnp.dot(num_acc[...] * rb, sel32_ref[...])


def _stage12(receivers, senders, edge_attribute, idx_f32_row, bases, nwins,
             W_target, W_source, W_edge, attn_mat, bcast_mat, bcast32,
             sel32):
    nblk = N_EDGES // CHUNK
    full = lambda shape: pl.BlockSpec(shape, lambda i, b, nw: (0, 0))
    edge_blk = lambda w: pl.BlockSpec((CHUNK, w), lambda i, b, nw: (i, 0))
    grid_spec = pltpu.PrefetchScalarGridSpec(
        num_scalar_prefetch=2,
        grid=(nblk,),
        in_specs=[
            edge_blk(IN_C), edge_blk(IN_C), edge_blk(IN_C),
            pl.BlockSpec((1, CHUNK), lambda i, b, nw: (0, i)),
            full((IN_C, HC)), full((IN_C, HC)), full((IN_C, HC)),
            full((HC, HEADS)), full((HEADS, HC)),
            full((HEADS, HC)), full((HC, OUT_C)),
        ],
        out_specs=[
            edge_blk(HC),
            pl.BlockSpec((NPAD, OUT_C), lambda i, b, nw: (0, 0)),
            pl.BlockSpec((NPAD, HEADS), lambda i, b, nw: (0, 0)),
        ],
        scratch_shapes=[
            pltpu.VMEM((NPAD, HC), jnp.float32),
            pltpu.VMEM((NPAD, HEADS), jnp.float32),
        ],
    )
    return pl.pallas_call(
        _s12_body,
        grid_spec=grid_spec,
        out_shape=[
            jax.ShapeDtypeStruct((N_EDGES, HC), jnp.float32),
            jax.ShapeDtypeStruct((NPAD, OUT_C), jnp.float32),
            jax.ShapeDtypeStruct((NPAD, HEADS), jnp.float32),
        ],
    )(bases, nwins, receivers, senders, edge_attribute, idx_f32_row,
      W_target, W_source, W_edge, attn_mat, bcast_mat, bcast32, sel32)


def _sc_gather(table_flat, idx_pad, per_tile):
    info = plsc.get_sparse_core_info()
    nc = info.num_cores
    ep = idx_pad.shape[0]
    tbl = table_flat.shape[0]

    mesh = plsc.VectorSubcoreMesh(core_axis_name="c", subcore_axis_name="s")

    @functools.partial(
        pl.kernel, mesh=mesh,
        out_type=jax.ShapeDtypeStruct((ep * HEADS,), jnp.float32),
        compiler_params=pltpu.CompilerParams(needs_layout_passes=False),
        scratch_types=[
            pltpu.VMEM((tbl,), jnp.float32),
            pltpu.VMEM((per_tile,), jnp.int32),
            pltpu.VMEM((per_tile * HEADS,), jnp.float32),
        ],
    )
    def k(table_hbm, idx_hbm, out_hbm, tbl_v, idx_v, out_v):
        wid = lax.axis_index("s") * nc + lax.axis_index("c")
        base = wid * per_tile
        pltpu.sync_copy(table_hbm, tbl_v)
        pltpu.sync_copy(idx_hbm.at[pl.ds(base, per_tile)], idx_v)
        lane = lax.iota(jnp.int32, 16) * HEADS

        def it(i, _):
            flat = idx_v[pl.ds(i * 16, 16)] * HEADS
            off = i * (16 * HEADS)
            for h in range(HEADS):
                g = plsc.load_gather(tbl_v, [flat + h])
                plsc.store_scatter(out_v, [lane + (off + h)], g)
            return 0

        lax.fori_loop(0, per_tile // 16, it, 0)
        pltpu.sync_copy(out_v, out_hbm.at[pl.ds(base * HEADS, per_tile * HEADS)])

    return k(table_flat, idx_pad)


EDGE_B4 = 3200


def _s4_body(p_ref, rp_ref, bm_ref, sel_ref, out_ref):
    dotf = functools.partial(jnp.dot, preferred_element_type=jnp.float32)
    rb = dotf(rp_ref[...].astype(jnp.bfloat16), bm_ref[...])
    out_ref[...] = dotf((p_ref[...] * rb).astype(jnp.bfloat16), sel_ref[...])


def _stage4(p, recip_pe, bcast_mat, sel_mat):
    nblk = N_EDGES // EDGE_B4
    return pl.pallas_call(
        _s4_body,
        grid=(nblk,),
        in_specs=[
            pl.BlockSpec((EDGE_B4, HC), lambda i: (i, 0)),
            pl.BlockSpec((EDGE_B4, HEADS), lambda i: (i, 0)),
            pl.BlockSpec((HEADS, HC), lambda i: (0, 0)),
            pl.BlockSpec((HC, OUT_C), lambda i: (0, 0)),
        ],
        out_specs=pl.BlockSpec((EDGE_B4, OUT_C), lambda i: (i, 0)),
        out_shape=jax.ShapeDtypeStruct((N_EDGES, OUT_C), jnp.float32),
    )(p, recip_pe, bcast_mat, sel_mat)


def kernel(receivers, senders, sender_idx, edge_attribute, W_source,
           W_target, W_edge, attn):
    idx = sender_idx.astype(jnp.int32)

    a0 = attn.reshape(HEADS, OUT_C)
    attn_mat = (a0[:, :, None] * jnp.eye(HEADS, dtype=jnp.float32)[:, None, :]
                ).reshape(HC, HEADS)
    bcast32 = (jnp.eye(HEADS, dtype=jnp.float32)[:, :, None]
               * jnp.ones((1, 1, OUT_C), jnp.float32)).reshape(HEADS, HC)
    bcast_mat = bcast32.astype(jnp.bfloat16)
    sel32 = jnp.tile(jnp.eye(OUT_C, dtype=jnp.float32) * (1.0 / HEADS),
                     (HEADS, 1))
    sel_mat = sel32.astype(jnp.bfloat16)
    wt16 = W_target.astype(jnp.bfloat16)
    ws16 = W_source.astype(jnp.bfloat16)
    we16 = W_edge.astype(jnp.bfloat16)

    starts = idx[::CHUNK]
    ends = idx[CHUNK - 1::CHUNK]
    bases = (starts // 8) * 8
    nwins = (ends - bases + WIN) // WIN

    idx_f32_row = idx.astype(jnp.float32).reshape(1, N_EDGES)

    p, aggr, recip = _stage12(
        receivers, senders, edge_attribute, idx_f32_row, bases, nwins,
        wt16, ws16, we16, attn_mat, bcast_mat, bcast32, sel32)

    ntiles = 32
    per_tile = -(-N_EDGES // ntiles)
    per_tile = -(-per_tile // 128) * 128
    ep = ntiles * per_tile
    idx_pad = jnp.pad(idx, (0, ep - N_EDGES))
    rp_flat = _sc_gather(recip.reshape(-1)[:N_NODES * HEADS], idx_pad, per_tile)
    rp = rp_flat.reshape(ep, HEADS)

    m_out = _stage4(p, rp, bcast_mat, sel_mat)
    return (aggr[:N_NODES], m_out)


# speedup vs baseline: 59.8474x; 1.0194x over previous
import functools

import jax
import jax.numpy as jnp
from jax import lax
from jax.experimental import pallas as pl
from jax.experimental.pallas import tpu as pltpu
from jax.experimental.pallas import tpu_sc as plsc

N_NODES = 10000
N_EDGES = 320000
IN_C = 128
OUT_C = 16
HEADS = 4
HC = OUT_C * HEADS


CHUNK = 6400
WIN = 128
NPAD = 10240


def _s12_body(base_ref, nwin_ref,
              r_ref, s_ref, ea_ref, idx_ref,
              wt_ref, ws_ref, we_ref, am_ref, bm_ref,
              bm32_ref, sel32_ref,
              p_ref, aggr_ref, recip_ref,
              num_acc, den_acc):
    i = pl.program_id(0)
    nsteps = pl.num_programs(0)

    @pl.when(i == 0)
    def _init():
        num_acc[...] = jnp.zeros_like(num_acc)
        den_acc[...] = jnp.zeros_like(den_acc)

    f32 = jnp.float32
    dotf = functools.partial(jnp.dot, preferred_element_type=f32)
    t = dotf(r_ref[...].astype(jnp.bfloat16), wt_ref[...])
    s = dotf(s_ref[...].astype(jnp.bfloat16), ws_ref[...])
    e = dotf(ea_ref[...].astype(jnp.bfloat16), we_ref[...])
    v = s + t + e
    v = jnp.where(v > 0, v, 0.01 * v)
    z = jnp.dot(v, am_ref[...])
    ez = jnp.exp(z)
    ez16 = ez.astype(jnp.bfloat16)
    p = t * jnp.dot(ez, bm32_ref[...])
    p16 = p.astype(jnp.bfloat16)
    p_ref[...] = p16

    base = base_ref[i]
    nwin = nwin_ref[i]
    idx_row = idx_ref[...]

    def win(w, _):
        w0 = base + w * WIN
        nid = w0.astype(jnp.float32) + lax.broadcasted_iota(
            jnp.int32, (WIN, 1), 0).astype(jnp.float32)
        oh = (idx_row == nid).astype(jnp.bfloat16)
        num_acc[pl.ds(w0, WIN), :] += dotf(oh, p16)
        den_acc[pl.ds(w0, WIN), :] += dotf(oh, ez16)
        return 0

    lax.fori_loop(0, nwin, win, 0)

    @pl.when(i == nsteps - 1)
    def _epilogue():
        recip = 1.0 / (den_acc[...] + 1e-16)
        recip_ref[...] = recip
        rb = jnp.dot(recip, bm32_ref[...])
        aggr_ref[...] = jnp.dot(num_acc[...] * rb, sel32_ref[...])


def _stage12(receivers, senders, edge_attribute, idx_f32_row, bases, nwins,
             W_target, W_source, W_edge, attn_mat, bcast_mat, bcast32,
             sel32):
    nblk = N_EDGES // CHUNK
    full = lambda shape: pl.BlockSpec(shape, lambda i, b, nw: (0, 0))
    edge_blk = lambda w: pl.BlockSpec((CHUNK, w), lambda i, b, nw: (i, 0))
    grid_spec = pltpu.PrefetchScalarGridSpec(
        num_scalar_prefetch=2,
        grid=(nblk,),
        in_specs=[
            edge_blk(IN_C), edge_blk(IN_C), edge_blk(IN_C),
            pl.BlockSpec((1, CHUNK), lambda i, b, nw: (0, i)),
            full((IN_C, HC)), full((IN_C, HC)), full((IN_C, HC)),
            full((HC, HEADS)), full((HEADS, HC)),
            full((HEADS, HC)), full((HC, OUT_C)),
        ],
        out_specs=[
            edge_blk(HC),
            pl.BlockSpec((NPAD, OUT_C), lambda i, b, nw: (0, 0)),
            pl.BlockSpec((NPAD, HEADS), lambda i, b, nw: (0, 0)),
        ],
        scratch_shapes=[
            pltpu.VMEM((NPAD, HC), jnp.float32),
            pltpu.VMEM((NPAD, HEADS), jnp.float32),
        ],
    )
    return pl.pallas_call(
        _s12_body,
        grid_spec=grid_spec,
        out_shape=[
            jax.ShapeDtypeStruct((N_EDGES, HC), jnp.bfloat16),
            jax.ShapeDtypeStruct((NPAD, OUT_C), jnp.float32),
            jax.ShapeDtypeStruct((NPAD, HEADS), jnp.float32),
        ],
    )(bases, nwins, receivers, senders, edge_attribute, idx_f32_row,
      W_target, W_source, W_edge, attn_mat, bcast_mat, bcast32, sel32)


def _sc_gather(table_flat, idx_pad, per_tile):
    info = plsc.get_sparse_core_info()
    nc = info.num_cores
    ep = idx_pad.shape[0]
    tbl = table_flat.shape[0]

    mesh = plsc.VectorSubcoreMesh(core_axis_name="c", subcore_axis_name="s")

    @functools.partial(
        pl.kernel, mesh=mesh,
        out_type=jax.ShapeDtypeStruct((ep * HEADS,), jnp.float32),
        compiler_params=pltpu.CompilerParams(needs_layout_passes=False),
        scratch_types=[
            pltpu.VMEM((tbl,), jnp.float32),
            pltpu.VMEM((per_tile,), jnp.int32),
            pltpu.VMEM((per_tile * HEADS,), jnp.float32),
        ],
    )
    def k(table_hbm, idx_hbm, out_hbm, tbl_v, idx_v, out_v):
        wid = lax.axis_index("s") * nc + lax.axis_index("c")
        base = wid * per_tile
        pltpu.sync_copy(table_hbm, tbl_v)
        pltpu.sync_copy(idx_hbm.at[pl.ds(base, per_tile)], idx_v)
        lane = lax.iota(jnp.int32, 16) * HEADS

        def it(i, _):
            flat = idx_v[pl.ds(i * 16, 16)] * HEADS
            off = i * (16 * HEADS)
            for h in range(HEADS):
                g = plsc.load_gather(tbl_v, [flat + h])
                plsc.store_scatter(out_v, [lane + (off + h)], g)
            return 0

        lax.fori_loop(0, per_tile // 16, it, 0)
        pltpu.sync_copy(out_v, out_hbm.at[pl.ds(base * HEADS, per_tile * HEADS)])

    return k(table_flat, idx_pad)


EDGE_B4 = 3200


def _s4_body(p_ref, rp_ref, bm_ref, sel_ref, out_ref):
    dotf = functools.partial(jnp.dot, preferred_element_type=jnp.float32)
    rb = dotf(rp_ref[...].astype(jnp.bfloat16), bm_ref[...])
    out_ref[...] = dotf((p_ref[...] * rb).astype(jnp.bfloat16), sel_ref[...])


def _stage4(p, recip_pe, bcast_mat, sel_mat):
    nblk = N_EDGES // EDGE_B4
    return pl.pallas_call(
        _s4_body,
        grid=(nblk,),
        in_specs=[
            pl.BlockSpec((EDGE_B4, HC), lambda i: (i, 0)),
            pl.BlockSpec((EDGE_B4, HEADS), lambda i: (i, 0)),
            pl.BlockSpec((HEADS, HC), lambda i: (0, 0)),
            pl.BlockSpec((HC, OUT_C), lambda i: (0, 0)),
        ],
        out_specs=pl.BlockSpec((EDGE_B4, OUT_C), lambda i: (i, 0)),
        out_shape=jax.ShapeDtypeStruct((N_EDGES, OUT_C), jnp.float32),
    )(p, recip_pe, bcast_mat, sel_mat)


def kernel(receivers, senders, sender_idx, edge_attribute, W_source,
           W_target, W_edge, attn):
    idx = sender_idx.astype(jnp.int32)

    a0 = attn.reshape(HEADS, OUT_C)
    attn_mat = (a0[:, :, None] * jnp.eye(HEADS, dtype=jnp.float32)[:, None, :]
                ).reshape(HC, HEADS)
    bcast32 = (jnp.eye(HEADS, dtype=jnp.float32)[:, :, None]
               * jnp.ones((1, 1, OUT_C), jnp.float32)).reshape(HEADS, HC)
    bcast_mat = bcast32.astype(jnp.bfloat16)
    sel32 = jnp.tile(jnp.eye(OUT_C, dtype=jnp.float32) * (1.0 / HEADS),
                     (HEADS, 1))
    sel_mat = sel32.astype(jnp.bfloat16)
    wt16 = W_target.astype(jnp.bfloat16)
    ws16 = W_source.astype(jnp.bfloat16)
    we16 = W_edge.astype(jnp.bfloat16)

    starts = idx[::CHUNK]
    ends = idx[CHUNK - 1::CHUNK]
    bases = (starts // 8) * 8
    nwins = (ends - bases + WIN) // WIN

    idx_f32_row = idx.astype(jnp.float32).reshape(1, N_EDGES)

    p, aggr, recip = _stage12(
        receivers, senders, edge_attribute, idx_f32_row, bases, nwins,
        wt16, ws16, we16, attn_mat, bcast_mat, bcast32, sel32)

    ntiles = 32
    per_tile = -(-N_EDGES // ntiles)
    per_tile = -(-per_tile // 128) * 128
    ep = ntiles * per_tile
    idx_pad = jnp.pad(idx, (0, ep - N_EDGES))
    rp_flat = _sc_gather(recip.reshape(-1)[:N_NODES * HEADS], idx_pad, per_tile)
    rp = rp_flat.reshape(ep, HEADS)

    m_out = _stage4(p, rp, bcast_mat, sel_mat)
    return (aggr[:N_NODES], m_out)


# speedup vs baseline: 62.9050x; 1.0511x over previous
import functools

import jax
import jax.numpy as jnp
from jax import lax
from jax.experimental import pallas as pl
from jax.experimental.pallas import tpu as pltpu
from jax.experimental.pallas import tpu_sc as plsc

N_NODES = 10000
N_EDGES = 320000
IN_C = 128
OUT_C = 16
HEADS = 4
HC = OUT_C * HEADS


CHUNK = 6400
WIN = 128
NPAD = 10240


def _s12_body(base_ref, nwin_ref,
              r_ref, s_ref, ea_ref, idx_ref,
              wt_ref, ws_ref, we_ref, am_ref, bm_ref,
              bm32_ref, sel32_ref, enum_ref, eden_ref,
              p_ref, aggr_ref, recip_ref,
              acc):
    i = pl.program_id(0)
    nsteps = pl.num_programs(0)

    @pl.when(i == 0)
    def _init():
        acc[...] = jnp.zeros_like(acc)

    f32 = jnp.float32
    dotf = functools.partial(jnp.dot, preferred_element_type=f32)
    t = dotf(r_ref[...].astype(jnp.bfloat16), wt_ref[...])
    s = dotf(s_ref[...].astype(jnp.bfloat16), ws_ref[...])
    e = dotf(ea_ref[...].astype(jnp.bfloat16), we_ref[...])
    v = s + t + e
    v = jnp.where(v > 0, v, 0.01 * v)
    z = jnp.dot(v, am_ref[...])
    ez = jnp.exp(z)
    ez16 = ez.astype(jnp.bfloat16)
    p = t * jnp.dot(ez, bm32_ref[...])
    p16 = p.astype(jnp.bfloat16)
    p_ref[...] = p16

    base = base_ref[i]
    nwin = nwin_ref[i]
    idx_row = idx_ref[...]
    rhs = jnp.concatenate([p16, ez16], axis=1)

    def win(w, _):
        w0 = base + w * WIN
        nid = w0.astype(jnp.float32) + lax.broadcasted_iota(
            jnp.int32, (WIN, 1), 0).astype(jnp.float32)
        oh = (idx_row == nid).astype(jnp.bfloat16)
        acc[pl.ds(w0, WIN), :] += dotf(oh, rhs)
        return 0

    lax.fori_loop(0, nwin, win, 0)

    @pl.when(i == nsteps - 1)
    def _epilogue():
        a = acc[...]
        den = jnp.dot(a, eden_ref[...])
        recip = 1.0 / (den + 1e-16)
        recip_ref[...] = recip
        num = jnp.dot(a, enum_ref[...])
        rb = jnp.dot(recip, bm32_ref[...])
        aggr_ref[...] = jnp.dot(num * rb, sel32_ref[...])


def _stage12(receivers, senders, edge_attribute, idx_f32_row, bases, nwins,
             W_target, W_source, W_edge, attn_mat, bcast_mat, bcast32,
             sel32, e_num, e_den):
    nblk = N_EDGES // CHUNK
    full = lambda shape: pl.BlockSpec(shape, lambda i, b, nw: (0, 0))
    edge_blk = lambda w: pl.BlockSpec((CHUNK, w), lambda i, b, nw: (i, 0))
    grid_spec = pltpu.PrefetchScalarGridSpec(
        num_scalar_prefetch=2,
        grid=(nblk,),
        in_specs=[
            edge_blk(IN_C), edge_blk(IN_C), edge_blk(IN_C),
            pl.BlockSpec((1, CHUNK), lambda i, b, nw: (0, i)),
            full((IN_C, HC)), full((IN_C, HC)), full((IN_C, HC)),
            full((HC, HEADS)), full((HEADS, HC)),
            full((HEADS, HC)), full((HC, OUT_C)),
            full((HC + HEADS, HC)), full((HC + HEADS, HEADS)),
        ],
        out_specs=[
            edge_blk(HC),
            pl.BlockSpec((NPAD, OUT_C), lambda i, b, nw: (0, 0)),
            pl.BlockSpec((NPAD, HEADS), lambda i, b, nw: (0, 0)),
        ],
        scratch_shapes=[
            pltpu.VMEM((NPAD, HC + HEADS), jnp.float32),
        ],
    )
    return pl.pallas_call(
        _s12_body,
        grid_spec=grid_spec,
        out_shape=[
            jax.ShapeDtypeStruct((N_EDGES, HC), jnp.bfloat16),
            jax.ShapeDtypeStruct((NPAD, OUT_C), jnp.float32),
            jax.ShapeDtypeStruct((NPAD, HEADS), jnp.float32),
        ],
    )(bases, nwins, receivers, senders, edge_attribute, idx_f32_row,
      W_target, W_source, W_edge, attn_mat, bcast_mat, bcast32, sel32,
      e_num, e_den)


def _sc_gather(table_flat, idx_pad, per_tile):
    info = plsc.get_sparse_core_info()
    nc = info.num_cores
    ep = idx_pad.shape[0]
    tbl = table_flat.shape[0]

    mesh = plsc.VectorSubcoreMesh(core_axis_name="c", subcore_axis_name="s")

    @functools.partial(
        pl.kernel, mesh=mesh,
        out_type=jax.ShapeDtypeStruct((ep * HEADS,), jnp.float32),
        compiler_params=pltpu.CompilerParams(needs_layout_passes=False),
        scratch_types=[
            pltpu.VMEM((tbl,), jnp.float32),
            pltpu.VMEM((per_tile,), jnp.int32),
            pltpu.VMEM((per_tile * HEADS,), jnp.float32),
        ],
    )
    def k(table_hbm, idx_hbm, out_hbm, tbl_v, idx_v, out_v):
        wid = lax.axis_index("s") * nc + lax.axis_index("c")
        base = wid * per_tile
        pltpu.sync_copy(table_hbm, tbl_v)
        pltpu.sync_copy(idx_hbm.at[pl.ds(base, per_tile)], idx_v)
        lane = lax.iota(jnp.int32, 16) * HEADS

        def it(i, _):
            flat = idx_v[pl.ds(i * 16, 16)] * HEADS
            off = i * (16 * HEADS)
            for h in range(HEADS):
                g = plsc.load_gather(tbl_v, [flat + h])
                plsc.store_scatter(out_v, [lane + (off + h)], g)
            return 0

        lax.fori_loop(0, per_tile // 16, it, 0)
        pltpu.sync_copy(out_v, out_hbm.at[pl.ds(base * HEADS, per_tile * HEADS)])

    return k(table_flat, idx_pad)


EDGE_B4 = 3200


def _s4_body(p_ref, rp_ref, bm_ref, sel_ref, out_ref):
    dotf = functools.partial(jnp.dot, preferred_element_type=jnp.float32)
    rb = dotf(rp_ref[...].astype(jnp.bfloat16), bm_ref[...])
    out_ref[...] = dotf((p_ref[...] * rb).astype(jnp.bfloat16), sel_ref[...])


def _stage4(p, recip_pe, bcast_mat, sel_mat):
    nblk = N_EDGES // EDGE_B4
    return pl.pallas_call(
        _s4_body,
        grid=(nblk,),
        in_specs=[
            pl.BlockSpec((EDGE_B4, HC), lambda i: (i, 0)),
            pl.BlockSpec((EDGE_B4, HEADS), lambda i: (i, 0)),
            pl.BlockSpec((HEADS, HC), lambda i: (0, 0)),
            pl.BlockSpec((HC, OUT_C), lambda i: (0, 0)),
        ],
        out_specs=pl.BlockSpec((EDGE_B4, OUT_C), lambda i: (i, 0)),
        out_shape=jax.ShapeDtypeStruct((N_EDGES, OUT_C), jnp.float32),
    )(p, recip_pe, bcast_mat, sel_mat)


def kernel(receivers, senders, sender_idx, edge_attribute, W_source,
           W_target, W_edge, attn):
    idx = sender_idx.astype(jnp.int32)

    a0 = attn.reshape(HEADS, OUT_C)
    attn_mat = (a0[:, :, None] * jnp.eye(HEADS, dtype=jnp.float32)[:, None, :]
                ).reshape(HC, HEADS)
    bcast32 = (jnp.eye(HEADS, dtype=jnp.float32)[:, :, None]
               * jnp.ones((1, 1, OUT_C), jnp.float32)).reshape(HEADS, HC)
    bcast_mat = bcast32.astype(jnp.bfloat16)
    sel32 = jnp.tile(jnp.eye(OUT_C, dtype=jnp.float32) * (1.0 / HEADS),
                     (HEADS, 1))
    sel_mat = sel32.astype(jnp.bfloat16)
    wt16 = W_target.astype(jnp.bfloat16)
    ws16 = W_source.astype(jnp.bfloat16)
    we16 = W_edge.astype(jnp.bfloat16)

    starts = idx[::CHUNK]
    ends = idx[CHUNK - 1::CHUNK]
    bases = (starts // 8) * 8
    nwins = (ends - bases + WIN) // WIN

    idx_f32_row = idx.astype(jnp.float32).reshape(1, N_EDGES)

    e_num = jnp.concatenate(
        [jnp.eye(HC, dtype=jnp.float32),
         jnp.zeros((HEADS, HC), jnp.float32)], axis=0)
    e_den = jnp.concatenate(
        [jnp.zeros((HC, HEADS), jnp.float32),
         jnp.eye(HEADS, dtype=jnp.float32)], axis=0)

    p, aggr, recip = _stage12(
        receivers, senders, edge_attribute, idx_f32_row, bases, nwins,
        wt16, ws16, we16, attn_mat, bcast_mat, bcast32, sel32,
        e_num, e_den)

    ntiles = 32
    per_tile = -(-N_EDGES // ntiles)
    per_tile = -(-per_tile // 128) * 128
    ep = ntiles * per_tile
    idx_pad = jnp.pad(idx, (0, ep - N_EDGES))
    rp_flat = _sc_gather(recip.reshape(-1)[:N_NODES * HEADS], idx_pad, per_tile)
    rp = rp_flat.reshape(ep, HEADS)

    m_out = _stage4(p, rp, bcast_mat, sel_mat)
    return (aggr[:N_NODES], m_out)
